# baseline (speedup 1.0000x reference)
import jax
import jax.numpy as jnp
from jax.experimental import pallas as pl
from jax.experimental.pallas import tpu as pltpu

_RATIO = 3


def _softplus(x):
    return jnp.maximum(x, 0.0) + jnp.log1p(jnp.exp(-jnp.abs(x)))


def _body(conf_ref, lab_ref, loc_ref, loct_ref, m4_ref,
          out_l_ref, out_c_ref, acc_ref):
    step = pl.program_id(0)
    nsteps = pl.num_programs(0)
    C = lab_ref.shape[0]

    @pl.when(step == 0)
    def _init():
        acc_ref[0] = 0.0
        acc_ref[1] = 0.0
        acc_ref[2] = 0.0

    c0 = conf_ref[0]
    c1 = conf_ref[1]
    lab = lab_ref[...]
    x = c1 - c0
    sp = _softplus(x)
    pos = lab > 0
    neg = lab == 0
    np_v = jnp.sum(pos.astype(jnp.int32), axis=(1, 2), keepdims=True)
    nneg_v = jnp.sum(neg.astype(jnp.int32), axis=(1, 2), keepdims=True)
    np_cl = jnp.maximum(np_v, 1)
    k_v = jnp.minimum(np_cl * _RATIO, nneg_v)

    ce_pos = jnp.sum(jnp.where(pos, sp - x, 0.0))
    s_all = jnp.sum(jnp.where(neg, sp, 0.0))

    d = loc_ref[...] - loct_ref[...]
    ad = jnp.abs(d)
    m = jnp.minimum(ad, 1.0)
    sl1 = m * (ad - 0.5 * m)
    ll = jnp.sum(jnp.where(m4_ref[...] != 0, sl1, 0.0))

    flags = jnp.logical_and(k_v < nneg_v, k_v > 0)
    nflag = jnp.sum(flags.astype(jnp.int32))

    acc_ref[0] = acc_ref[0] + ll
    acc_ref[1] = acc_ref[1] + ce_pos + s_all
    acc_ref[2] = acc_ref[2] + jnp.sum(np_cl).astype(jnp.float32)

    @pl.when(nflag > 0)
    def _slow():
        def row(r, tot):
            labr = lab_ref[r]
            xr = conf_ref[1, r] - conf_ref[0, r]
            spr = _softplus(xr)
            negr = labr == 0
            np_r = jnp.sum((labr > 0).astype(jnp.int32))
            nneg_r = jnp.sum(negr.astype(jnp.int32))
            k_r = jnp.minimum(jnp.maximum(np_r, 1) * _RATIO, nneg_r)
            flag_r = jnp.logical_and(k_r < nneg_r, k_r > 0)
            s_all_r = jnp.sum(jnp.where(negr, spr, 0.0))
            u = jnp.where(negr,
                          jax.lax.bitcast_convert_type(spr, jnp.int32),
                          jnp.int32(-1))
            p = jnp.int32(0)
            for i in range(30, -1, -1):
                cand = p | jnp.int32(1 << i)
                cnt = jnp.sum((u >= cand).astype(jnp.int32))
                p = jnp.where(cnt >= k_r, cand, p)
            gt = u > p
            cnt_gt = jnp.sum(gt.astype(jnp.int32))
            sum_gt = jnp.sum(jnp.where(gt, spr, 0.0))
            tval = jax.lax.bitcast_convert_type(p, jnp.float32)
            topk = sum_gt + (k_r - cnt_gt).astype(jnp.float32) * tval
            return tot + jnp.where(flag_r, topk - s_all_r, 0.0)

        fix = jax.lax.fori_loop(0, C, row, 0.0)
        acc_ref[1] = acc_ref[1] + fix

    @pl.when(step == nsteps - 1)
    def _fin():
        npf = acc_ref[2]
        out_l_ref[0, 0] = acc_ref[0] / npf
        out_c_ref[0, 0] = acc_ref[1] / npf


def kernel(player_loc, player_conf, player_loc_t, player_conf_t):
    B, P = player_conf_t.shape
    rows_c = -(-P // 128)
    ppad = rows_c * 128 - P
    rows_l = (P * 4) // 128
    C = 16 if B % 16 == 0 else (8 if B % 8 == 0 else 1)

    confp = jnp.pad(player_conf.transpose(2, 0, 1),
                    ((0, 0), (0, 0), (0, ppad))).reshape(2, B, rows_c, 128)
    labp = jnp.pad(player_conf_t, ((0, 0), (0, ppad)),
                   constant_values=-1).reshape(B, rows_c, 128)
    locv = player_loc.reshape(B, rows_l, 128)
    loctv = player_loc_t.reshape(B, rows_l, 128)
    m4 = jnp.repeat((player_conf_t > 0).astype(jnp.int8), 4,
                    axis=1).reshape(B, rows_l, 128)

    row = lambda i: (i, 0, 0)
    out_l, out_c = pl.pallas_call(
        _body,
        grid=(B // C,),
        in_specs=[
            pl.BlockSpec((2, C, rows_c, 128), lambda i: (0, i, 0, 0)),
            pl.BlockSpec((C, rows_c, 128), row),
            pl.BlockSpec((C, rows_l, 128), row),
            pl.BlockSpec((C, rows_l, 128), row),
            pl.BlockSpec((C, rows_l, 128), row),
        ],
        out_specs=[
            pl.BlockSpec(memory_space=pltpu.SMEM),
            pl.BlockSpec(memory_space=pltpu.SMEM),
        ],
        out_shape=[
            jax.ShapeDtypeStruct((1, 1), jnp.float32),
            jax.ShapeDtypeStruct((1, 1), jnp.float32),
        ],
        scratch_shapes=[
            pltpu.SMEM((3,), jnp.float32),
        ],
        compiler_params=pltpu.CompilerParams(
            dimension_semantics=("arbitrary",),
        ),
    )(confp, labp, locv, loctv, m4)
    return (out_l[0, 0], out_c[0, 0])


# speedup vs baseline: 1.3122x; 1.3122x over previous
import jax
import jax.numpy as jnp
from jax.experimental import pallas as pl
from jax.experimental.pallas import tpu as pltpu

_RATIO = 3


def _softplus(x):
    return jnp.maximum(x, 0.0) + jnp.log1p(jnp.exp(-jnp.abs(x)))


def _deint_mat(n):
    r = jax.lax.broadcasted_iota(jnp.int32, (2 * n, n), 0)
    c = jax.lax.broadcasted_iota(jnp.int32, (2 * n, n), 1)
    return jnp.where(r == 2 * c, -1.0,
                     jnp.where(r == 2 * c + 1, 1.0, 0.0)).astype(jnp.float32)


def _exp4_mat(n):
    r = jax.lax.broadcasted_iota(jnp.int32, (n, 4 * n), 0)
    c = jax.lax.broadcasted_iota(jnp.int32, (n, 4 * n), 1)
    return (c // 4 == r).astype(jnp.float32)


def _mm(a, b):
    return jax.lax.dot_general(a, b, (((1,), (0,)), ((), ())),
                               preferred_element_type=jnp.float32)


def _body(v_ref, lab_ref, loc_ref, loct_ref, out_l_ref, out_c_ref, acc_ref):
    step = pl.program_id(0)
    nsteps = pl.num_programs(0)
    C, R, W = lab_ref.shape
    P = R * W

    @pl.when(step == 0)
    def _init():
        acc_ref[0] = 0.0
        acc_ref[1] = 0.0
        acc_ref[2] = 0.0

    v2 = v_ref[...].reshape(C * R, 2 * W)
    x = _mm(v2, _deint_mat(W))
    sp = _softplus(x)
    labf = lab_ref[...].astype(jnp.float32)
    lab2 = labf.reshape(C * R, W)

    np_f = jnp.sum(labf, axis=(1, 2), keepdims=True)
    np_v = np_f.astype(jnp.int32)
    nneg_v = P - np_v
    np_cl = jnp.maximum(np_v, 1)
    k_v = jnp.minimum(np_cl * _RATIO, nneg_v)

    conf_sum = jnp.sum(sp) - jnp.sum(lab2 * x)

    d = loc_ref[...] - loct_ref[...]
    ad = jnp.abs(d)
    m = jnp.minimum(ad, 1.0)
    sl1 = m * (ad - 0.5 * m)
    mF = _mm(lab2, _exp4_mat(W)).reshape(C, R, 4 * W)
    ll = jnp.sum(mF * sl1)

    flags = jnp.logical_and(k_v < nneg_v, k_v > 0)
    nflag = jnp.sum(flags.astype(jnp.int32))

    acc_ref[0] = acc_ref[0] + ll
    acc_ref[1] = acc_ref[1] + conf_sum
    acc_ref[2] = acc_ref[2] + jnp.sum(np_cl).astype(jnp.float32)

    @pl.when(nflag > 0)
    def _slow():
        def row(r, tot):
            labr = lab_ref[r]
            vr = v_ref[r].reshape(R, 2 * W)
            xr = _mm(vr, _deint_mat(W))
            spr = _softplus(xr)
            negr = labr == 0
            np_r = jnp.sum(labr)
            nneg_r = P - np_r
            k_r = jnp.minimum(jnp.maximum(np_r, 1) * _RATIO, nneg_r)
            flag_r = jnp.logical_and(k_r < nneg_r, k_r > 0)
            s_all_r = jnp.sum(jnp.where(negr, spr, 0.0))
            u = jnp.where(negr,
                          jax.lax.bitcast_convert_type(spr, jnp.int32),
                          jnp.int32(-1))
            p = jnp.int32(0)
            for i in range(30, -1, -1):
                cand = p | jnp.int32(1 << i)
                cnt = jnp.sum((u >= cand).astype(jnp.int32))
                p = jnp.where(cnt >= k_r, cand, p)
            gt = u > p
            cnt_gt = jnp.sum(gt.astype(jnp.int32))
            sum_gt = jnp.sum(jnp.where(gt, spr, 0.0))
            tval = jax.lax.bitcast_convert_type(p, jnp.float32)
            topk = sum_gt + (k_r - cnt_gt).astype(jnp.float32) * tval
            return tot + jnp.where(flag_r, topk - s_all_r, 0.0)

        fix = jax.lax.fori_loop(0, C, row, 0.0)
        acc_ref[1] = acc_ref[1] + fix

    @pl.when(step == nsteps - 1)
    def _fin():
        npf = acc_ref[2]
        out_l_ref[0, 0] = acc_ref[0] / npf
        out_c_ref[0, 0] = acc_ref[1] / npf


def kernel(player_loc, player_conf, player_loc_t, player_conf_t):
    B, P = player_conf_t.shape
    R = 100
    W = P // R
    C = 8 if B % 8 == 0 else 1

    v = player_conf.reshape(B, R, 2 * W)
    lab = player_conf_t.reshape(B, R, W)
    locv = player_loc.reshape(B, R, 4 * W)
    loctv = player_loc_t.reshape(B, R, 4 * W)

    row = lambda i: (i, 0, 0)
    out_l, out_c = pl.pallas_call(
        _body,
        grid=(B // C,),
        in_specs=[
            pl.BlockSpec((C, R, 2 * W), row),
            pl.BlockSpec((C, R, W), row),
            pl.BlockSpec((C, R, 4 * W), row),
            pl.BlockSpec((C, R, 4 * W), row),
        ],
        out_specs=[
            pl.BlockSpec(memory_space=pltpu.SMEM),
            pl.BlockSpec(memory_space=pltpu.SMEM),
        ],
        out_shape=[
            jax.ShapeDtypeStruct((1, 1), jnp.float32),
            jax.ShapeDtypeStruct((1, 1), jnp.float32),
        ],
        scratch_shapes=[
            pltpu.SMEM((3,), jnp.float32),
        ],
        compiler_params=pltpu.CompilerParams(
            dimension_semantics=("arbitrary",),
        ),
    )(v, lab, locv, loctv)
    return (out_l[0, 0], out_c[0, 0])


# speedup vs baseline: 1.3377x; 1.0194x over previous
import jax
import jax.numpy as jnp
from jax.experimental import pallas as pl
from jax.experimental.pallas import tpu as pltpu

_RATIO = 3


def _softplus(x):
    return jnp.maximum(x, 0.0) + jnp.log1p(jnp.exp(-jnp.abs(x)))


def _deint_mat(n):
    r = jax.lax.broadcasted_iota(jnp.int32, (2 * n, n), 0)
    c = jax.lax.broadcasted_iota(jnp.int32, (2 * n, n), 1)
    return jnp.where(r == 2 * c, -1.0,
                     jnp.where(r == 2 * c + 1, 1.0, 0.0)).astype(jnp.float32)


def _exp4_mat(n):
    r = jax.lax.broadcasted_iota(jnp.int32, (n, 4 * n), 0)
    c = jax.lax.broadcasted_iota(jnp.int32, (n, 4 * n), 1)
    return (c // 4 == r).astype(jnp.float32)


def _mm(a, b):
    return jax.lax.dot_general(a, b, (((1,), (0,)), ((), ())),
                               preferred_element_type=jnp.float32)


def _body(v_ref, lab_ref, loc_ref, loct_ref, out_l_ref, out_c_ref, acc_ref):
    step = pl.program_id(0)
    nsteps = pl.num_programs(0)
    C, R, W = lab_ref.shape
    P = R * W

    @pl.when(step == 0)
    def _init():
        acc_ref[0] = 0.0
        acc_ref[1] = 0.0
        acc_ref[2] = 0.0

    v2 = v_ref[...].reshape(C * R, 2 * W)
    x = _mm(v2, _deint_mat(W))
    sp = _softplus(x)
    labf = lab_ref[...].astype(jnp.float32)
    lab2 = labf.reshape(C * R, W)

    np_f = jnp.sum(labf, axis=(1, 2), keepdims=True)
    np_v = np_f.astype(jnp.int32)
    nneg_v = P - np_v
    np_cl = jnp.maximum(np_v, 1)
    k_v = jnp.minimum(np_cl * _RATIO, nneg_v)

    conf_sum = jnp.sum(sp) - jnp.sum(lab2 * x)

    d = loc_ref[...] - loct_ref[...]
    ad = jnp.abs(d)
    m = jnp.minimum(ad, 1.0)
    sl1 = m * (ad - 0.5 * m)
    mF = _mm(lab2, _exp4_mat(W)).reshape(C, R, 4 * W)
    ll = jnp.sum(mF * sl1)

    flags = jnp.logical_and(k_v < nneg_v, k_v > 0)
    nflag = jnp.sum(flags.astype(jnp.int32))

    acc_ref[0] = acc_ref[0] + ll
    acc_ref[1] = acc_ref[1] + conf_sum
    acc_ref[2] = acc_ref[2] + jnp.sum(np_cl).astype(jnp.float32)

    @pl.when(nflag > 0)
    def _slow():
        def row(r, tot):
            labr = lab_ref[r]
            vr = v_ref[r].reshape(R, 2 * W)
            xr = _mm(vr, _deint_mat(W))
            spr = _softplus(xr)
            negr = labr == 0
            np_r = jnp.sum(labr)
            nneg_r = P - np_r
            k_r = jnp.minimum(jnp.maximum(np_r, 1) * _RATIO, nneg_r)
            flag_r = jnp.logical_and(k_r < nneg_r, k_r > 0)
            s_all_r = jnp.sum(jnp.where(negr, spr, 0.0))
            u = jnp.where(negr,
                          jax.lax.bitcast_convert_type(spr, jnp.int32),
                          jnp.int32(-1))
            p = jnp.int32(0)
            for i in range(30, -1, -1):
                cand = p | jnp.int32(1 << i)
                cnt = jnp.sum((u >= cand).astype(jnp.int32))
                p = jnp.where(cnt >= k_r, cand, p)
            gt = u > p
            cnt_gt = jnp.sum(gt.astype(jnp.int32))
            sum_gt = jnp.sum(jnp.where(gt, spr, 0.0))
            tval = jax.lax.bitcast_convert_type(p, jnp.float32)
            topk = sum_gt + (k_r - cnt_gt).astype(jnp.float32) * tval
            return tot + jnp.where(flag_r, topk - s_all_r, 0.0)

        fix = jax.lax.fori_loop(0, C, row, 0.0)
        acc_ref[1] = acc_ref[1] + fix

    @pl.when(step == nsteps - 1)
    def _fin():
        npf = acc_ref[2]
        out_l_ref[0, 0] = acc_ref[0] / npf
        out_c_ref[0, 0] = acc_ref[1] / npf


def kernel(player_loc, player_conf, player_loc_t, player_conf_t):
    B, P = player_conf_t.shape
    R = 100
    W = P // R
    C = 16 if B % 16 == 0 else (8 if B % 8 == 0 else 1)

    v = player_conf.reshape(B, R, 2 * W)
    lab = player_conf_t.reshape(B, R, W)
    locv = player_loc.reshape(B, R, 4 * W)
    loctv = player_loc_t.reshape(B, R, 4 * W)

    row = lambda i: (i, 0, 0)
    out_l, out_c = pl.pallas_call(
        _body,
        grid=(B // C,),
        in_specs=[
            pl.BlockSpec((C, R, 2 * W), row),
            pl.BlockSpec((C, R, W), row),
            pl.BlockSpec((C, R, 4 * W), row),
            pl.BlockSpec((C, R, 4 * W), row),
        ],
        out_specs=[
            pl.BlockSpec(memory_space=pltpu.SMEM),
            pl.BlockSpec(memory_space=pltpu.SMEM),
        ],
        out_shape=[
            jax.ShapeDtypeStruct((1, 1), jnp.float32),
            jax.ShapeDtypeStruct((1, 1), jnp.float32),
        ],
        scratch_shapes=[
            pltpu.SMEM((3,), jnp.float32),
        ],
        compiler_params=pltpu.CompilerParams(
            dimension_semantics=("arbitrary",),
        ),
    )(v, lab, locv, loctv)
    return (out_l[0, 0], out_c[0, 0])
